# baseline (device time: 58670 ns/iter reference)
import jax
import jax.numpy as jnp
from jax import lax
from jax.experimental import pallas as pl
from jax.experimental.pallas import tpu as pltpu

N_DEV = 16
M = 1024
N = 1024
CHUNK = M // N_DEV


def kernel(A, B):
    def body(a_ref, b_ref, out_ref, partial_ref, rs_buf,
             rs_send, rs_recv, ag_send, ag_recv):
        me = lax.axis_index("i")

        a = a_ref[...].astype(jnp.bfloat16)
        b = b_ref[...].astype(jnp.bfloat16)
        partial = jnp.dot(a, b, preferred_element_type=jnp.float32)
        partial_ref[...] = partial.astype(jnp.bfloat16)

        rs_rdmas = []
        for d in range(1, N_DEV):
            t = lax.rem(me + d, N_DEV)
            rdma = pltpu.make_async_remote_copy(
                src_ref=partial_ref.at[pl.ds(t * CHUNK, CHUNK), :],
                dst_ref=rs_buf.at[d - 1],
                send_sem=rs_send.at[d - 1],
                recv_sem=rs_recv.at[d - 1],
                device_id=(t,),
                device_id_type=pl.DeviceIdType.MESH,
            )
            rdma.start()
            rs_rdmas.append(rdma)

        for rdma in rs_rdmas:
            rdma.wait_recv()

        own = partial_ref[pl.ds(me * CHUNK, CHUNK), :].astype(jnp.float32)
        z = own + jnp.sum(rs_buf[...].astype(jnp.float32), axis=0)
        silu = (z / (1.0 + jnp.exp(-z))).astype(jnp.bfloat16)
        out_ref[pl.ds(me * CHUNK, CHUNK), :] = silu

        ag_rdmas = []
        for d in range(1, N_DEV):
            t = lax.rem(me + d, N_DEV)
            rdma = pltpu.make_async_remote_copy(
                src_ref=out_ref.at[pl.ds(me * CHUNK, CHUNK), :],
                dst_ref=out_ref.at[pl.ds(me * CHUNK, CHUNK), :],
                send_sem=ag_send.at[d - 1],
                recv_sem=ag_recv.at[d - 1],
                device_id=(t,),
                device_id_type=pl.DeviceIdType.MESH,
            )
            rdma.start()
            ag_rdmas.append(rdma)

        for rdma in ag_rdmas:
            rdma.wait_recv()
        for rdma in rs_rdmas:
            rdma.wait_send()
        for rdma in ag_rdmas:
            rdma.wait_send()

    return pl.pallas_call(
        body,
        out_shape=jax.ShapeDtypeStruct((M, N), jnp.bfloat16),
        in_specs=[
            pl.BlockSpec(memory_space=pltpu.VMEM),
            pl.BlockSpec(memory_space=pltpu.VMEM),
        ],
        out_specs=pl.BlockSpec(memory_space=pltpu.VMEM),
        scratch_shapes=[
            pltpu.VMEM((M, N), jnp.bfloat16),
            pltpu.VMEM((N_DEV - 1, CHUNK, N), jnp.bfloat16),
            pltpu.SemaphoreType.DMA((N_DEV - 1,)),
            pltpu.SemaphoreType.DMA((N_DEV - 1,)),
            pltpu.SemaphoreType.DMA((N_DEV - 1,)),
            pltpu.SemaphoreType.DMA((N_DEV - 1,)),
        ],
    )(A, B)
